# baseline (speedup 1.0000x reference)
import functools
import math

import jax
import jax.numpy as jnp
from jax import lax
from jax.experimental import pallas as pl
from jax.experimental.pallas import tpu as pltpu
from jax.experimental.pallas import tpu_sc as plsc

VOCAB = 100000
D = 1024
SEQ = 4096
BATCH = 4
B = BATCH * SEQ
SCALE = math.sqrt(D)

NC = 2
NS = 16
NW = NC * NS
NSEQ_W = SEQ // NW
ROWS_PER_W = B // NW
CK = 16
NCHUNK = ROWS_PER_W // CK
NGRP = NCHUNK // BATCH
NBUF = 4
L = 16
VPR = D // L

_mesh = plsc.VectorSubcoreMesh(
    core_axis_name="c", subcore_axis_name="s", num_cores=NC, num_subcores=NS
)


@functools.partial(
    pl.kernel,
    out_type=jax.ShapeDtypeStruct((B, D), jnp.float32),
    mesh=_mesh,
    scratch_types=[
        pltpu.VMEM((ROWS_PER_W,), jnp.int32),
        pltpu.VMEM((NBUF * CK, D), jnp.float32),
        pltpu.VMEM((2 * CK, D), jnp.float32),
        pltpu.SemaphoreType.DMA,
        pltpu.SemaphoreType.DMA,
        pltpu.SemaphoreType.DMA,
        pltpu.SemaphoreType.DMA,
        pltpu.SemaphoreType.DMA,
        pltpu.SemaphoreType.DMA,
        pltpu.SemaphoreType.DMA,
    ],
)
def _embed_sc(x_hbm, tok_hbm, pos_hbm, out_hbm, idx_v, rows_v, pos_v,
              g0, g1, g2, g3, p0, p1, ssem):
    gsems = (g0, g1, g2, g3)
    psems = (p0, p1)
    wid = lax.axis_index("s") * NC + lax.axis_index("c")
    s0 = wid * NSEQ_W

    for b in range(BATCH):
        pltpu.sync_copy(
            x_hbm.at[pl.ds(b * SEQ + s0, NSEQ_W)],
            idx_v.at[pl.ds(b * NSEQ_W, NSEQ_W)],
        )

    def idx_off(c):
        return (c % BATCH) * NSEQ_W + (c // BATCH) * CK

    def out_off(c):
        return (c % BATCH) * SEQ + s0 + (c // BATCH) * CK

    def issue_gather(c, buf):
        pltpu.async_copy(
            tok_hbm.at[idx_v.at[pl.ds(idx_off(c), CK)]],
            rows_v.at[pl.ds(buf * CK, CK)],
            gsems[buf],
        )

    def wait_gather(buf):
        pltpu.make_async_copy(
            tok_hbm.at[pl.ds(0, CK)],
            rows_v.at[pl.ds(buf * CK, CK)],
            gsems[buf],
        ).wait()

    def issue_pos(q, half):
        pltpu.async_copy(
            pos_hbm.at[pl.ds(s0 + q * CK, CK)],
            pos_v.at[pl.ds(half * CK, CK)],
            psems[half],
        )

    def wait_pos(half):
        pltpu.make_async_copy(
            pos_hbm.at[pl.ds(0, CK)],
            pos_v.at[pl.ds(half * CK, CK)],
            psems[half],
        ).wait()

    def drain_store():
        pltpu.make_async_copy(
            rows_v.at[pl.ds(0, CK)], out_hbm.at[pl.ds(0, CK)], ssem
        ).wait()

    def compute(buf, poff):
        @pl.loop(0, CK)
        def _row(r):
            for j in range(VPR):
                sl = pl.ds(j * L, L)
                rows_v[buf * CK + r, sl] = (
                    rows_v[buf * CK + r, sl] * SCALE + pos_v[poff + r, sl]
                )

    def store(c, buf):
        pltpu.async_copy(
            rows_v.at[pl.ds(buf * CK, CK)],
            out_hbm.at[pl.ds(out_off(c), CK)],
            ssem,
        )

    issue_pos(0, 0)
    for c in range(NBUF - 1):
        issue_gather(c, c)

    @pl.loop(0, NGRP)
    def _grp(q):
        poff = (q % 2) * CK
        @pl.when(q % 2 == 0)
        def _():
            wait_pos(0)

            @pl.when(q < NGRP - 1)
            def _():
                issue_pos(q + 1, 1)

        @pl.when(q % 2 == 1)
        def _():
            wait_pos(1)

            @pl.when(q < NGRP - 1)
            def _():
                issue_pos(q + 1, 0)

        for i in range(BATCH):
            c = BATCH * q + i

            @pl.when(c + NBUF - 1 < NCHUNK)
            def _():
                @pl.when(c >= 1)
                def _():
                    drain_store()

                issue_gather(c + NBUF - 1, (i + NBUF - 1) % NBUF)

            wait_gather(i)
            compute(i, poff)
            store(c, i)

    for _ in range(NBUF):
        drain_store()


def kernel(x, token_table, pos_table):
    out = _embed_sc(x.reshape(-1), token_table, pos_table)
    return out.reshape(BATCH, SEQ, D)


# speedup vs baseline: 1.3587x; 1.3587x over previous
import functools
import math

import jax
import jax.numpy as jnp
from jax import lax
from jax.experimental import pallas as pl
from jax.experimental.pallas import tpu as pltpu
from jax.experimental.pallas import tpu_sc as plsc

VOCAB = 100000
D = 1024
SEQ = 4096
BATCH = 4
B = BATCH * SEQ
SCALE = math.sqrt(D)

NC = 2
NS = 16
NW = NC * NS
NSEQ_W = SEQ // NW
ROWS_PER_W = B // NW
CK = 16
NCHUNK = ROWS_PER_W // CK
NGRP = NCHUNK // BATCH
NBUF = 4
L = 16
VPR = D // L

_mesh = plsc.VectorSubcoreMesh(
    core_axis_name="c", subcore_axis_name="s", num_cores=NC, num_subcores=NS
)


@functools.partial(
    pl.kernel,
    out_type=jax.ShapeDtypeStruct((B, D), jnp.float32),
    mesh=_mesh,
    scratch_types=[
        pltpu.VMEM((ROWS_PER_W,), jnp.int32),
        pltpu.VMEM((NBUF * CK, D), jnp.float32),
        pltpu.VMEM((2 * CK, D), jnp.float32),
        pltpu.SemaphoreType.DMA,
        pltpu.SemaphoreType.DMA,
        pltpu.SemaphoreType.DMA,
        pltpu.SemaphoreType.DMA,
        pltpu.SemaphoreType.DMA,
        pltpu.SemaphoreType.DMA,
        pltpu.SemaphoreType.DMA,
    ],
)
def _embed_sc(x_hbm, tok_hbm, pos_hbm, out_hbm, idx_v, rows_v, pos_v,
              g0, g1, g2, g3, p0, p1, ssem):
    gsems = (g0, g1, g2, g3)
    psems = (p0, p1)
    wid = lax.axis_index("s") * NC + lax.axis_index("c")
    s0 = wid * NSEQ_W

    for b in range(BATCH):
        pltpu.sync_copy(
            x_hbm.at[pl.ds(b * SEQ + s0, NSEQ_W)],
            idx_v.at[pl.ds(b * NSEQ_W, NSEQ_W)],
        )

    def idx_off(c):
        return (c % BATCH) * NSEQ_W + (c // BATCH) * CK

    def out_off(c):
        return (c % BATCH) * SEQ + s0 + (c // BATCH) * CK

    def issue_gather(c, buf):
        pltpu.async_copy(
            tok_hbm.at[idx_v.at[pl.ds(idx_off(c), CK)]],
            rows_v.at[pl.ds(buf * CK, CK)],
            gsems[buf],
        )

    def wait_gather(buf):
        pltpu.make_async_copy(
            tok_hbm.at[pl.ds(0, CK)],
            rows_v.at[pl.ds(buf * CK, CK)],
            gsems[buf],
        ).wait()

    def issue_pos(q, half):
        pltpu.async_copy(
            pos_hbm.at[pl.ds(s0 + q * CK, CK)],
            pos_v.at[pl.ds(half * CK, CK)],
            psems[half],
        )

    def wait_pos(half):
        pltpu.make_async_copy(
            pos_hbm.at[pl.ds(0, CK)],
            pos_v.at[pl.ds(half * CK, CK)],
            psems[half],
        ).wait()

    def drain_store():
        pltpu.make_async_copy(
            rows_v.at[pl.ds(0, CK)], out_hbm.at[pl.ds(0, CK)], ssem
        ).wait()

    def compute(buf, half):
        boff = buf * CK
        poff = half * CK
        @pl.loop(0, CK)
        def _row(r):
            for j in range(VPR):
                sl = pl.ds(j * L, L)
                rows_v[boff + r, sl] = (
                    rows_v[boff + r, sl] * SCALE + pos_v[poff + r, sl]
                )

    def store(c, buf):
        pltpu.async_copy(
            rows_v.at[pl.ds(buf * CK, CK)],
            out_hbm.at[pl.ds(out_off(c), CK)],
            ssem,
        )

    issue_pos(0, 0)
    issue_pos(1, 1)
    for c in range(NBUF - 1):
        issue_gather(c, c)

    @pl.loop(0, NGRP // 2)
    def _pair(t):
        for qq in range(2):
            q = 2 * t + qq
            wait_pos(qq)
            for i in range(BATCH):
                c = BATCH * q + i

                @pl.when(c + NBUF - 1 < NCHUNK)
                def _():
                    @pl.when(c >= 1)
                    def _():
                        drain_store()

                    issue_gather(c + NBUF - 1, (i + NBUF - 1) % NBUF)

                wait_gather(i)
                compute(i, qq)
                store(c, i)
            @pl.when(q + 2 < NGRP)
            def _():
                issue_pos(q + 2, qq)

    for _ in range(NBUF):
        drain_store()


def kernel(x, token_table, pos_table):
    out = _embed_sc(x.reshape(-1), token_table, pos_table)
    return out.reshape(BATCH, SEQ, D)
